# baseline (speedup 1.0000x reference)
import functools

import jax
import jax.numpy as jnp
from jax import lax
from jax.experimental import pallas as pl
from jax.experimental.pallas import tpu as pltpu
from jax.experimental.pallas import tpu_sc as plsc

N = 10000
D = 128
HD = D // 2
NPAD = 10240
SHARD = NPAD // 16
CH = 96
EPAD = 331776
NROW2D = EPAD // CH
NCH = EPAD // 16 // CH

_SC_PARAMS = pltpu.CompilerParams(
    needs_layout_passes=False, use_tc_tiling_on_sc=False)


def _rsqrt_nr(v):
    s = 0.5 * (1.0 + v)
    for _ in range(24):
        s = 0.5 * (s + v / s)
    return 1.0 / s


def _zero_rows(buf, nrow):
    def zr(i, _):
        for j in range(HD // 16):
            buf[i, pl.ds(j * 16, 16)] = jnp.zeros((16,), jnp.float32)
        return 0
    lax.fori_loop(0, nrow, zr, 0)


def _scale_rows(rows, nrm_row):
    def srow(rr, _):
        for q in range(4):
            r = rr * 4 + q
            nsp = plsc.load_gather(nrm_row, [jnp.full((16,), r, jnp.int32)])
            for j in range(HD // 16):
                rows[r, pl.ds(j * 16, 16)] = rows[r, pl.ds(j * 16, 16)] * nsp
        return 0
    lax.fori_loop(0, CH // 4, srow, 0)


def _agg_loop(feat_h, srcb, dstb, nrmb, rows0, rows1, g0, g1, acc_sh,
              double=False):
    if double:
        def pair(g, _):
            d0 = pltpu.async_copy(feat_h.at[srcb.at[2 * g]], rows0, g0)
            d1 = pltpu.async_copy(feat_h.at[srcb.at[2 * g + 1]], rows1, g1)
            d0.wait()
            _scale_rows(rows0, nrmb.at[2 * g])
            pltpu.sync_copy(rows0, acc_sh.at[dstb.at[2 * g]], add=True)
            d1.wait()
            _scale_rows(rows1, nrmb.at[2 * g + 1])
            pltpu.sync_copy(rows1, acc_sh.at[dstb.at[2 * g + 1]], add=True)
            return 0
        lax.fori_loop(0, NCH // 2, pair, 0)
    else:
        def chunk(ch, _):
            pltpu.async_copy(feat_h.at[srcb.at[ch]], rows0, g0).wait()
            _scale_rows(rows0, nrmb.at[ch])
            pltpu.sync_copy(rows0, acc_sh.at[dstb.at[ch]], add=True)
            return 0
        lax.fori_loop(0, NCH, chunk, 0)


def _sc_pass_a(src2d, dst2d, w2d, feat4):
    mesh = plsc.VectorSubcoreMesh(core_axis_name="c", subcore_axis_name="s")

    @functools.partial(
        pl.kernel,
        out_type=[
            jax.ShapeDtypeStruct((4 * NPAD, HD), jnp.float32),
            jax.ShapeDtypeStruct((NROW2D, CH), jnp.float32),
            jax.ShapeDtypeStruct((NPAD,), jnp.float32),
        ],
        mesh=mesh,
        compiler_params=_SC_PARAMS,
        scratch_types=[
            pltpu.VMEM((NCH, CH), jnp.int32),
            pltpu.VMEM((NCH, CH), jnp.int32),
            pltpu.VMEM((NCH, CH), jnp.float32),
            pltpu.VMEM((CH, HD), jnp.float32),
            pltpu.VMEM((CH, HD), jnp.float32),
            pltpu.VMEM((128, HD), jnp.float32),
            pltpu.VMEM((SHARD,), jnp.float32),
            pltpu.VMEM((NPAD,), jnp.float32),
            pltpu.VMEM_SHARED((NPAD, HD), jnp.float32),
            pltpu.VMEM_SHARED((NPAD,), jnp.float32),
            pltpu.SemaphoreType.DMA,
            pltpu.SemaphoreType.DMA,
        ],
    )
    def k(src_h, dst_h, w_h, feat_h, out_h, norm_h, dis_h,
          srcb, dstb, wb, rows0, rows1, zb, stg, dis_t,
          acc_sh, deg_sh, g0, g1):
        cid = lax.axis_index("c")
        sid = lax.axis_index("s")

        pltpu.sync_copy(src_h.at[pl.ds(sid * NCH, NCH)], srcb)
        pltpu.sync_copy(dst_h.at[pl.ds(sid * NCH, NCH)], dstb)
        pltpu.sync_copy(w_h.at[pl.ds(sid * NCH, NCH)], wb)

        _zero_rows(zb, 128)

        def zstg(i, _):
            stg[pl.ds(i * 16, 16)] = jnp.zeros((16,), jnp.float32)
            return 0
        lax.fori_loop(0, SHARD // 16, zstg, 0)

        for kb in range(SHARD // 128):
            pltpu.sync_copy(zb, acc_sh.at[pl.ds(sid * SHARD + kb * 128, 128)])
        pltpu.sync_copy(stg, deg_sh.at[pl.ds(sid * SHARD, SHARD)])
        plsc.subcore_barrier()

        def degc(ch, _):
            pltpu.sync_copy(wb.at[ch], deg_sh.at[dstb.at[ch]], add=True)
            return 0
        lax.fori_loop(0, NCH, degc, 0)
        plsc.subcore_barrier()

        pltpu.sync_copy(deg_sh.at[pl.ds(sid * SHARD, SHARD)], stg)

        def disc(i, _):
            v = stg[pl.ds(i * 16, 16)] + 1.0
            stg[pl.ds(i * 16, 16)] = _rsqrt_nr(v)
            return 0
        lax.fori_loop(0, SHARD // 16, disc, 0)
        pltpu.sync_copy(stg, deg_sh.at[pl.ds(sid * SHARD, SHARD)])
        plsc.subcore_barrier()
        pltpu.sync_copy(deg_sh, dis_t)

        @pl.when(jnp.logical_and(cid == 0, sid == 0))
        def _():
            pltpu.sync_copy(dis_t, dis_h)

        def normc(ch, _):
            for j in range(CH // 16):
                s16 = srcb[ch, pl.ds(j * 16, 16)]
                d16 = dstb[ch, pl.ds(j * 16, 16)]
                w16 = wb[ch, pl.ds(j * 16, 16)]
                nrm = plsc.load_gather(dis_t, [s16]) * w16 * plsc.load_gather(dis_t, [d16])
                wb[ch, pl.ds(j * 16, 16)] = nrm
            return 0
        lax.fori_loop(0, NCH, normc, 0)

        @pl.when(cid == 0)
        def _():
            pltpu.sync_copy(wb, norm_h.at[pl.ds(sid * NCH, NCH)])

        def offc(delta):
            def f(ch, _):
                for j in range(CH // 16):
                    srcb[ch, pl.ds(j * 16, 16)] = srcb[ch, pl.ds(j * 16, 16)] + delta
                return 0
            lax.fori_loop(0, NCH, f, 0)

        offc(cid * N)

        for phase in range(2):
            if phase == 1:
                offc(2 * N)
            _agg_loop(feat_h, srcb, dstb, wb, rows0, rows1, g0, g1, acc_sh)
            plsc.subcore_barrier()
            pltpu.sync_copy(
                acc_sh.at[pl.ds(sid * SHARD, SHARD)],
                out_h.at[pl.ds((2 * phase + cid) * NPAD + sid * SHARD, SHARD)])
            if phase == 0:
                for kb in range(SHARD // 128):
                    pltpu.sync_copy(zb, acc_sh.at[pl.ds(sid * SHARD + kb * 128, 128)])
                plsc.subcore_barrier()

    return k(src2d, dst2d, w2d, feat4)


def _sc_pass_b(src2d, dst2d, norm2d, rh2):
    mesh = plsc.VectorSubcoreMesh(core_axis_name="c", subcore_axis_name="s")

    @functools.partial(
        pl.kernel,
        out_type=[
            jax.ShapeDtypeStruct((2 * NPAD, HD), jnp.float32),
        ],
        mesh=mesh,
        compiler_params=_SC_PARAMS,
        scratch_types=[
            pltpu.VMEM((NCH, CH), jnp.int32),
            pltpu.VMEM((NCH, CH), jnp.int32),
            pltpu.VMEM((NCH, CH), jnp.float32),
            pltpu.VMEM((CH, HD), jnp.float32),
            pltpu.VMEM((CH, HD), jnp.float32),
            pltpu.VMEM((128, HD), jnp.float32),
            pltpu.VMEM_SHARED((NPAD, HD), jnp.float32),
            pltpu.SemaphoreType.DMA,
            pltpu.SemaphoreType.DMA,
        ],
    )
    def k(src_h, dst_h, norm_h, rh_h, out_h,
          srcb, dstb, normb, rows0, rows1, zb, acc_sh, g0, g1):
        cid = lax.axis_index("c")
        sid = lax.axis_index("s")

        pltpu.sync_copy(src_h.at[pl.ds(sid * NCH, NCH)], srcb)
        pltpu.sync_copy(dst_h.at[pl.ds(sid * NCH, NCH)], dstb)
        pltpu.sync_copy(norm_h.at[pl.ds(sid * NCH, NCH)], normb)

        _zero_rows(zb, 128)
        for kb in range(SHARD // 128):
            pltpu.sync_copy(zb, acc_sh.at[pl.ds(sid * SHARD + kb * 128, 128)])

        def offc(ch, _):
            for j in range(CH // 16):
                srcb[ch, pl.ds(j * 16, 16)] = srcb[ch, pl.ds(j * 16, 16)] + cid * N
            return 0
        lax.fori_loop(0, NCH, offc, 0)
        plsc.subcore_barrier()

        _agg_loop(rh_h, srcb, dstb, normb, rows0, rows1, g0, g1, acc_sh,
                  double=True)
        plsc.subcore_barrier()
        pltpu.sync_copy(acc_sh.at[pl.ds(sid * SHARD, SHARD)],
                        out_h.at[pl.ds(cid * NPAD + sid * SHARD, SHARD)])

    return k(src2d, dst2d, norm2d, rh2)


def _tc_gates(saggx, saggh, x, h, dis2, w12, b12, w3a, b3):
    BR = 1000

    def body(sx, sh, xb, hb, db, w12r, b12r, w3ar, b3r, u_o, rh_o, cp_o):
        invd = db[...] * db[...]
        aggx = sx[...] + xb[...] * invd
        aggh = sh[...] + hb[...] * invd
        aggc = jnp.concatenate([aggx, aggh], axis=1)
        ru = jax.nn.sigmoid(
            jnp.dot(aggc, w12r[...], preferred_element_type=jnp.float32) + b12r[...])
        u_o[...] = ru[:, D:]
        rh_o[...] = ru[:, :D] * hb[...]
        cp_o[...] = jnp.dot(aggx, w3ar[...], preferred_element_type=jnp.float32) + b3r[...]

    bs_row = pl.BlockSpec((BR, D), lambda i: (i, 0))
    bs_col = pl.BlockSpec((BR, 1), lambda i: (i, 0))

    def full(shape):
        return pl.BlockSpec(shape, lambda i: (0, 0))

    return pl.pallas_call(
        body,
        grid=(N // BR,),
        in_specs=[bs_row, bs_row, bs_row, bs_row, bs_col,
                  full((2 * D, 2 * D)), full((1, 2 * D)), full((D, D)), full((1, D))],
        out_specs=[bs_row, bs_row, bs_row],
        out_shape=[jax.ShapeDtypeStruct((N, D), jnp.float32)] * 3,
    )(saggx, saggh, x, h, dis2, w12, b12, w3a, b3)


def _tc_final(srh, rh, dis2, cpart, u, h, w3b):
    BR = 1000

    def body(sb, rhb, db, cpb, ub, hb, w3br, o):
        invd = db[...] * db[...]
        aggrh = sb[...] + rhb[...] * invd
        c = jnp.tanh(cpb[...] + jnp.dot(aggrh, w3br[...],
                                        preferred_element_type=jnp.float32))
        o[...] = ub[...] * hb[...] + (1.0 - ub[...]) * c

    bs_row = pl.BlockSpec((BR, D), lambda i: (i, 0))
    bs_col = pl.BlockSpec((BR, 1), lambda i: (i, 0))

    return pl.pallas_call(
        body,
        grid=(N // BR,),
        in_specs=[bs_row, bs_row, bs_col, bs_row, bs_row, bs_row,
                  pl.BlockSpec((D, D), lambda i: (0, 0))],
        out_specs=bs_row,
        out_shape=jax.ShapeDtypeStruct((N, D), jnp.float32),
    )(srh, rh, dis2, cpart, u, h, w3b)


def kernel(x, edge_index, edge_weight, h, W1, b1, W2, b2, W3, b3):
    src = edge_index[0].astype(jnp.int32)
    dst = edge_index[1].astype(jnp.int32)
    w = edge_weight.astype(jnp.float32)
    pad = EPAD - src.shape[0]
    src2 = jnp.concatenate([src, jnp.zeros((pad,), jnp.int32)]).reshape(NROW2D, CH)
    dst2 = jnp.concatenate([dst, jnp.zeros((pad,), jnp.int32)]).reshape(NROW2D, CH)
    w2 = jnp.concatenate([w, jnp.zeros((pad,), jnp.float32)]).reshape(NROW2D, CH)
    feat4 = jnp.concatenate([x[:, :HD], x[:, HD:], h[:, :HD], h[:, HD:]], axis=0)

    out_a, norm2, dis = _sc_pass_a(src2, dst2, w2, feat4)
    saggx = jnp.concatenate([out_a[0:N], out_a[NPAD:NPAD + N]], axis=1)
    saggh = jnp.concatenate([out_a[2 * NPAD:2 * NPAD + N],
                             out_a[3 * NPAD:3 * NPAD + N]], axis=1)
    dis2 = dis[0:N].reshape(N, 1)

    w12 = jnp.concatenate([W1, W2], axis=1)
    b12 = jnp.concatenate([b1, b2]).reshape(1, 2 * D)
    u, rh, cpart = _tc_gates(saggx, saggh, x, h, dis2, w12, b12,
                             W3[:D], b3.reshape(1, D))

    rh2 = jnp.concatenate([rh[:, :HD], rh[:, HD:]], axis=0)
    out_b, = _sc_pass_b(src2, dst2, norm2, rh2)
    srh = jnp.concatenate([out_b[0:N], out_b[NPAD:NPAD + N]], axis=1)
    return _tc_final(srh, rh, dis2, cpart, u, h, W3[D:])


# speedup vs baseline: 1.1934x; 1.1934x over previous
import functools

import jax
import jax.numpy as jnp
from jax import lax
from jax.experimental import pallas as pl
from jax.experimental.pallas import tpu as pltpu
from jax.experimental.pallas import tpu_sc as plsc

N = 10000
D = 128
HD = D // 2
NPAD = 10240
SHARD = NPAD // 16
CH = 128
EPAD = 327680
NROW2D = EPAD // CH
NCH = EPAD // 16 // CH

_SC_PARAMS = pltpu.CompilerParams(
    needs_layout_passes=False, use_tc_tiling_on_sc=False)


def _rsqrt_nr(v):
    s = 0.5 * (1.0 + v)
    for _ in range(24):
        s = 0.5 * (s + v / s)
    return 1.0 / s


def _zero_rows(buf, nrow):
    def zr(i, _):
        for j in range(HD // 16):
            buf[i, pl.ds(j * 16, 16)] = jnp.zeros((16,), jnp.float32)
        return 0
    lax.fori_loop(0, nrow, zr, 0)


def _scale_rows(rows, nrm_row):
    def srow(rr, _):
        for q in range(4):
            r = rr * 4 + q
            nsp = plsc.load_gather(nrm_row, [jnp.full((16,), r, jnp.int32)])
            for j in range(HD // 16):
                rows[r, pl.ds(j * 16, 16)] = rows[r, pl.ds(j * 16, 16)] * nsp
        return 0
    lax.fori_loop(0, CH // 4, srow, 0)


def _agg_loop(feat_h, srcb, dstb, nrmb, rows0, rows1, g0, g1, acc_sh,
              double=False):
    if double:
        def pair(g, _):
            d0 = pltpu.async_copy(feat_h.at[srcb.at[2 * g]], rows0, g0)
            d1 = pltpu.async_copy(feat_h.at[srcb.at[2 * g + 1]], rows1, g1)
            d0.wait()
            _scale_rows(rows0, nrmb.at[2 * g])
            pltpu.sync_copy(rows0, acc_sh.at[dstb.at[2 * g]], add=True)
            d1.wait()
            _scale_rows(rows1, nrmb.at[2 * g + 1])
            pltpu.sync_copy(rows1, acc_sh.at[dstb.at[2 * g + 1]], add=True)
            return 0
        lax.fori_loop(0, NCH // 2, pair, 0)
    else:
        def chunk(ch, _):
            pltpu.async_copy(feat_h.at[srcb.at[ch]], rows0, g0).wait()
            _scale_rows(rows0, nrmb.at[ch])
            pltpu.sync_copy(rows0, acc_sh.at[dstb.at[ch]], add=True)
            return 0
        lax.fori_loop(0, NCH, chunk, 0)


def _sc_pass_a(src2d, dst2d, w2d, feat4):
    mesh = plsc.VectorSubcoreMesh(core_axis_name="c", subcore_axis_name="s")

    @functools.partial(
        pl.kernel,
        out_type=[
            jax.ShapeDtypeStruct((4 * NPAD, HD), jnp.float32),
            jax.ShapeDtypeStruct((NROW2D, CH), jnp.float32),
            jax.ShapeDtypeStruct((NPAD,), jnp.float32),
        ],
        mesh=mesh,
        compiler_params=_SC_PARAMS,
        scratch_types=[
            pltpu.VMEM((NCH, CH), jnp.int32),
            pltpu.VMEM((NCH, CH), jnp.int32),
            pltpu.VMEM((NCH, CH), jnp.float32),
            pltpu.VMEM((CH, HD), jnp.float32),
            pltpu.VMEM((CH, HD), jnp.float32),
            pltpu.VMEM((128, HD), jnp.float32),
            pltpu.VMEM((SHARD,), jnp.float32),
            pltpu.VMEM((NPAD,), jnp.float32),
            pltpu.VMEM_SHARED((NPAD, HD), jnp.float32),
            pltpu.VMEM_SHARED((NPAD,), jnp.float32),
            pltpu.SemaphoreType.DMA,
            pltpu.SemaphoreType.DMA,
        ],
    )
    def k(src_h, dst_h, w_h, feat_h, out_h, norm_h, dis_h,
          srcb, dstb, wb, rows0, rows1, zb, stg, dis_t,
          acc_sh, deg_sh, g0, g1):
        cid = lax.axis_index("c")
        sid = lax.axis_index("s")

        pltpu.sync_copy(src_h.at[pl.ds(sid * NCH, NCH)], srcb)
        pltpu.sync_copy(dst_h.at[pl.ds(sid * NCH, NCH)], dstb)
        pltpu.sync_copy(w_h.at[pl.ds(sid * NCH, NCH)], wb)

        _zero_rows(zb, 128)

        def zstg(i, _):
            stg[pl.ds(i * 16, 16)] = jnp.zeros((16,), jnp.float32)
            return 0
        lax.fori_loop(0, SHARD // 16, zstg, 0)

        for kb in range(SHARD // 128):
            pltpu.sync_copy(zb, acc_sh.at[pl.ds(sid * SHARD + kb * 128, 128)])
        pltpu.sync_copy(stg, deg_sh.at[pl.ds(sid * SHARD, SHARD)])
        plsc.subcore_barrier()

        def degc(g, _):
            ds = [pltpu.async_copy(wb.at[g * 8 + q], deg_sh.at[dstb.at[g * 8 + q]],
                                   g0, add=True) for q in range(8)]
            for d in ds:
                d.wait()
            return 0
        lax.fori_loop(0, NCH // 8, degc, 0)
        plsc.subcore_barrier()

        pltpu.sync_copy(deg_sh.at[pl.ds(sid * SHARD, SHARD)], stg)

        def disc(i, _):
            v = stg[pl.ds(i * 16, 16)] + 1.0
            stg[pl.ds(i * 16, 16)] = _rsqrt_nr(v)
            return 0
        lax.fori_loop(0, SHARD // 16, disc, 0)
        pltpu.sync_copy(stg, deg_sh.at[pl.ds(sid * SHARD, SHARD)])
        plsc.subcore_barrier()
        pltpu.sync_copy(deg_sh, dis_t)

        @pl.when(jnp.logical_and(cid == 0, sid == 0))
        def _():
            pltpu.sync_copy(dis_t, dis_h)

        def normc(ch, _):
            for j in range(CH // 16):
                s16 = srcb[ch, pl.ds(j * 16, 16)]
                d16 = dstb[ch, pl.ds(j * 16, 16)]
                w16 = wb[ch, pl.ds(j * 16, 16)]
                nrm = plsc.load_gather(dis_t, [s16]) * w16 * plsc.load_gather(dis_t, [d16])
                wb[ch, pl.ds(j * 16, 16)] = nrm
            return 0
        lax.fori_loop(0, NCH, normc, 0)

        @pl.when(cid == 0)
        def _():
            pltpu.sync_copy(wb, norm_h.at[pl.ds(sid * NCH, NCH)])

        def offc(delta):
            def f(ch, _):
                for j in range(CH // 16):
                    srcb[ch, pl.ds(j * 16, 16)] = srcb[ch, pl.ds(j * 16, 16)] + delta
                return 0
            lax.fori_loop(0, NCH, f, 0)

        offc(cid * N)

        for phase in range(2):
            if phase == 1:
                offc(2 * N)
            _agg_loop(feat_h, srcb, dstb, wb, rows0, rows1, g0, g1, acc_sh)
            plsc.subcore_barrier()
            pltpu.sync_copy(
                acc_sh.at[pl.ds(sid * SHARD, SHARD)],
                out_h.at[pl.ds((2 * phase + cid) * NPAD + sid * SHARD, SHARD)])
            if phase == 0:
                for kb in range(SHARD // 128):
                    pltpu.sync_copy(zb, acc_sh.at[pl.ds(sid * SHARD + kb * 128, 128)])
                plsc.subcore_barrier()

    return k(src2d, dst2d, w2d, feat4)


def _sc_pass_b(src2d, dst2d, norm2d, rh2):
    mesh = plsc.VectorSubcoreMesh(core_axis_name="c", subcore_axis_name="s")

    @functools.partial(
        pl.kernel,
        out_type=[
            jax.ShapeDtypeStruct((2 * NPAD, HD), jnp.float32),
        ],
        mesh=mesh,
        compiler_params=_SC_PARAMS,
        scratch_types=[
            pltpu.VMEM((NCH, CH), jnp.int32),
            pltpu.VMEM((NCH, CH), jnp.int32),
            pltpu.VMEM((NCH, CH), jnp.float32),
            pltpu.VMEM((CH, HD), jnp.float32),
            pltpu.VMEM((CH, HD), jnp.float32),
            pltpu.VMEM((128, HD), jnp.float32),
            pltpu.VMEM_SHARED((NPAD, HD), jnp.float32),
            pltpu.SemaphoreType.DMA,
            pltpu.SemaphoreType.DMA,
        ],
    )
    def k(src_h, dst_h, norm_h, rh_h, out_h,
          srcb, dstb, normb, rows0, rows1, zb, acc_sh, g0, g1):
        cid = lax.axis_index("c")
        sid = lax.axis_index("s")

        pltpu.sync_copy(src_h.at[pl.ds(sid * NCH, NCH)], srcb)
        pltpu.sync_copy(dst_h.at[pl.ds(sid * NCH, NCH)], dstb)
        pltpu.sync_copy(norm_h.at[pl.ds(sid * NCH, NCH)], normb)

        _zero_rows(zb, 128)
        for kb in range(SHARD // 128):
            pltpu.sync_copy(zb, acc_sh.at[pl.ds(sid * SHARD + kb * 128, 128)])

        def offc(ch, _):
            for j in range(CH // 16):
                srcb[ch, pl.ds(j * 16, 16)] = srcb[ch, pl.ds(j * 16, 16)] + cid * N
            return 0
        lax.fori_loop(0, NCH, offc, 0)
        plsc.subcore_barrier()

        _agg_loop(rh_h, srcb, dstb, normb, rows0, rows1, g0, g1, acc_sh)
        plsc.subcore_barrier()
        pltpu.sync_copy(acc_sh.at[pl.ds(sid * SHARD, SHARD)],
                        out_h.at[pl.ds(cid * NPAD + sid * SHARD, SHARD)])

    return k(src2d, dst2d, norm2d, rh2)


def _tc_gates(saggx, saggh, x, h, dis2, w12, b12, w3a, b3):
    BR = 1000

    def body(sx, sh, xb, hb, db, w12r, b12r, w3ar, b3r, u_o, rh_o, cp_o):
        invd = db[...] * db[...]
        aggx = sx[...] + xb[...] * invd
        aggh = sh[...] + hb[...] * invd
        aggc = jnp.concatenate([aggx, aggh], axis=1)
        ru = jax.nn.sigmoid(
            jnp.dot(aggc, w12r[...], preferred_element_type=jnp.float32) + b12r[...])
        u_o[...] = ru[:, D:]
        rh_o[...] = ru[:, :D] * hb[...]
        cp_o[...] = jnp.dot(aggx, w3ar[...], preferred_element_type=jnp.float32) + b3r[...]

    bs_row = pl.BlockSpec((BR, D), lambda i: (i, 0))
    bs_col = pl.BlockSpec((BR, 1), lambda i: (i, 0))

    def full(shape):
        return pl.BlockSpec(shape, lambda i: (0, 0))

    return pl.pallas_call(
        body,
        grid=(N // BR,),
        in_specs=[bs_row, bs_row, bs_row, bs_row, bs_col,
                  full((2 * D, 2 * D)), full((1, 2 * D)), full((D, D)), full((1, D))],
        out_specs=[bs_row, bs_row, bs_row],
        out_shape=[jax.ShapeDtypeStruct((N, D), jnp.float32)] * 3,
    )(saggx, saggh, x, h, dis2, w12, b12, w3a, b3)


def _tc_final(srh, rh, dis2, cpart, u, h, w3b):
    BR = 1000

    def body(sb, rhb, db, cpb, ub, hb, w3br, o):
        invd = db[...] * db[...]
        aggrh = sb[...] + rhb[...] * invd
        c = jnp.tanh(cpb[...] + jnp.dot(aggrh, w3br[...],
                                        preferred_element_type=jnp.float32))
        o[...] = ub[...] * hb[...] + (1.0 - ub[...]) * c

    bs_row = pl.BlockSpec((BR, D), lambda i: (i, 0))
    bs_col = pl.BlockSpec((BR, 1), lambda i: (i, 0))

    return pl.pallas_call(
        body,
        grid=(N // BR,),
        in_specs=[bs_row, bs_row, bs_col, bs_row, bs_row, bs_row,
                  pl.BlockSpec((D, D), lambda i: (0, 0))],
        out_specs=bs_row,
        out_shape=jax.ShapeDtypeStruct((N, D), jnp.float32),
    )(srh, rh, dis2, cpart, u, h, w3b)


def kernel(x, edge_index, edge_weight, h, W1, b1, W2, b2, W3, b3):
    src = edge_index[0].astype(jnp.int32)
    dst = edge_index[1].astype(jnp.int32)
    w = edge_weight.astype(jnp.float32)
    pad = EPAD - src.shape[0]
    src2 = jnp.concatenate([src, jnp.zeros((pad,), jnp.int32)]).reshape(NROW2D, CH)
    dst2 = jnp.concatenate([dst, jnp.zeros((pad,), jnp.int32)]).reshape(NROW2D, CH)
    w2 = jnp.concatenate([w, jnp.zeros((pad,), jnp.float32)]).reshape(NROW2D, CH)
    feat4 = jnp.concatenate([x[:, :HD], x[:, HD:], h[:, :HD], h[:, HD:]], axis=0)

    out_a, norm2, dis = _sc_pass_a(src2, dst2, w2, feat4)
    saggx = jnp.concatenate([out_a[0:N], out_a[NPAD:NPAD + N]], axis=1)
    saggh = jnp.concatenate([out_a[2 * NPAD:2 * NPAD + N],
                             out_a[3 * NPAD:3 * NPAD + N]], axis=1)
    dis2 = dis[0:N].reshape(N, 1)

    w12 = jnp.concatenate([W1, W2], axis=1)
    b12 = jnp.concatenate([b1, b2]).reshape(1, 2 * D)
    u, rh, cpart = _tc_gates(saggx, saggh, x, h, dis2, w12, b12,
                             W3[:D], b3.reshape(1, D))

    rh2 = jnp.concatenate([rh[:, :HD], rh[:, HD:]], axis=0)
    out_b, = _sc_pass_b(src2, dst2, norm2, rh2)
    srh = jnp.concatenate([out_b[0:N], out_b[NPAD:NPAD + N]], axis=1)
    return _tc_final(srh, rh, dis2, cpart, u, h, W3[D:])


# speedup vs baseline: 1.1991x; 1.0048x over previous
import functools

import jax
import jax.numpy as jnp
from jax import lax
from jax.experimental import pallas as pl
from jax.experimental.pallas import tpu as pltpu
from jax.experimental.pallas import tpu_sc as plsc

N = 10000
D = 128
HD = D // 2
NPAD = 10240
SHARD = NPAD // 16
CH = 128
EPAD = 327680
NROW2D = EPAD // CH
NCH = EPAD // 16 // CH

_SC_PARAMS = pltpu.CompilerParams(
    needs_layout_passes=False, use_tc_tiling_on_sc=False)


def _rsqrt_nr(v):
    s = 0.5 * (1.0 + v)
    for _ in range(24):
        s = 0.5 * (s + v / s)
    return 1.0 / s


def _zero_rows(buf, nrow):
    def zr(i, _):
        for j in range(HD // 16):
            buf[i, pl.ds(j * 16, 16)] = jnp.zeros((16,), jnp.float32)
        return 0
    lax.fori_loop(0, nrow, zr, 0)


def _scale_rows(rows, nrm_row):
    def srow(rr, _):
        for q in range(4):
            r = rr * 4 + q
            nsp = plsc.load_gather(nrm_row, [jnp.full((16,), r, jnp.int32)])
            for j in range(HD // 16):
                rows[r, pl.ds(j * 16, 16)] = rows[r, pl.ds(j * 16, 16)] * nsp
        return 0
    lax.fori_loop(0, CH // 4, srow, 0)


def _agg_loop(feat_h, srcb, dstb, nrmb, rows, g0, acc_sh):
    def chunk(ch, _):
        pltpu.async_copy(feat_h.at[srcb.at[ch]], rows, g0).wait()
        _scale_rows(rows, nrmb.at[ch])
        pltpu.sync_copy(rows, acc_sh.at[dstb.at[ch]], add=True)
        return 0
    lax.fori_loop(0, NCH, chunk, 0)


def _sc_pass_a(src2d, dst2d, w2d, feat4):
    mesh = plsc.VectorSubcoreMesh(core_axis_name="c", subcore_axis_name="s")

    @functools.partial(
        pl.kernel,
        out_type=[
            jax.ShapeDtypeStruct((4 * NPAD, HD), jnp.float32),
            jax.ShapeDtypeStruct((NROW2D, CH), jnp.float32),
            jax.ShapeDtypeStruct((NPAD,), jnp.float32),
        ],
        mesh=mesh,
        compiler_params=_SC_PARAMS,
        scratch_types=[
            pltpu.VMEM((NCH, CH), jnp.int32),
            pltpu.VMEM((NCH, CH), jnp.int32),
            pltpu.VMEM((NCH, CH), jnp.float32),
            pltpu.VMEM((CH, HD), jnp.float32),
            pltpu.VMEM((128, HD), jnp.float32),
            pltpu.VMEM((SHARD,), jnp.float32),
            pltpu.VMEM((NPAD,), jnp.float32),
            pltpu.VMEM_SHARED((NPAD, HD), jnp.float32),
            pltpu.VMEM_SHARED((NPAD,), jnp.float32),
            pltpu.SemaphoreType.DMA,
        ],
    )
    def k(src_h, dst_h, w_h, feat_h, out_h, norm_h, dis_h,
          srcb, dstb, wb, rows, zb, stg, dis_t,
          acc_sh, deg_sh, g0):
        cid = lax.axis_index("c")
        sid = lax.axis_index("s")

        pltpu.sync_copy(src_h.at[pl.ds(sid * NCH, NCH)], srcb)
        pltpu.sync_copy(dst_h.at[pl.ds(sid * NCH, NCH)], dstb)
        pltpu.sync_copy(w_h.at[pl.ds(sid * NCH, NCH)], wb)

        _zero_rows(zb, 128)

        def zstg(i, _):
            stg[pl.ds(i * 16, 16)] = jnp.zeros((16,), jnp.float32)
            return 0
        lax.fori_loop(0, SHARD // 16, zstg, 0)

        for kb in range(SHARD // 128):
            pltpu.sync_copy(zb, acc_sh.at[pl.ds(sid * SHARD + kb * 128, 128)])
        pltpu.sync_copy(stg, deg_sh.at[pl.ds(sid * SHARD, SHARD)])
        plsc.subcore_barrier()

        def degc(g, _):
            ds = [pltpu.async_copy(wb.at[g * 8 + q], deg_sh.at[dstb.at[g * 8 + q]],
                                   g0, add=True) for q in range(8)]
            for d in ds:
                d.wait()
            return 0
        lax.fori_loop(0, NCH // 8, degc, 0)
        plsc.subcore_barrier()

        pltpu.sync_copy(deg_sh.at[pl.ds(sid * SHARD, SHARD)], stg)

        def disc(i, _):
            v = stg[pl.ds(i * 16, 16)] + 1.0
            stg[pl.ds(i * 16, 16)] = _rsqrt_nr(v)
            return 0
        lax.fori_loop(0, SHARD // 16, disc, 0)
        pltpu.sync_copy(stg, deg_sh.at[pl.ds(sid * SHARD, SHARD)])
        plsc.subcore_barrier()
        pltpu.sync_copy(deg_sh, dis_t)

        @pl.when(jnp.logical_and(cid == 0, sid == 0))
        def _():
            pltpu.sync_copy(dis_t, dis_h)

        def normc(ch, _):
            for j in range(CH // 16):
                s16 = srcb[ch, pl.ds(j * 16, 16)]
                d16 = dstb[ch, pl.ds(j * 16, 16)]
                w16 = wb[ch, pl.ds(j * 16, 16)]
                nrm = plsc.load_gather(dis_t, [s16]) * w16 * plsc.load_gather(dis_t, [d16])
                wb[ch, pl.ds(j * 16, 16)] = nrm
            return 0
        lax.fori_loop(0, NCH, normc, 0)

        @pl.when(cid == 0)
        def _():
            pltpu.sync_copy(wb, norm_h.at[pl.ds(sid * NCH, NCH)])

        def offc(delta):
            def f(ch, _):
                for j in range(CH // 16):
                    srcb[ch, pl.ds(j * 16, 16)] = srcb[ch, pl.ds(j * 16, 16)] + delta
                return 0
            lax.fori_loop(0, NCH, f, 0)

        offc(cid * N)

        for phase in range(2):
            if phase == 1:
                offc(2 * N)
            _agg_loop(feat_h, srcb, dstb, wb, rows, g0, acc_sh)
            plsc.subcore_barrier()
            pltpu.sync_copy(
                acc_sh.at[pl.ds(sid * SHARD, SHARD)],
                out_h.at[pl.ds((2 * phase + cid) * NPAD + sid * SHARD, SHARD)])
            if phase == 0:
                for kb in range(SHARD // 128):
                    pltpu.sync_copy(zb, acc_sh.at[pl.ds(sid * SHARD + kb * 128, 128)])
                plsc.subcore_barrier()

    return k(src2d, dst2d, w2d, feat4)


def _sc_pass_b(src2d, dst2d, norm2d, rh2):
    mesh = plsc.VectorSubcoreMesh(core_axis_name="c", subcore_axis_name="s")

    @functools.partial(
        pl.kernel,
        out_type=[
            jax.ShapeDtypeStruct((2 * NPAD, HD), jnp.float32),
        ],
        mesh=mesh,
        compiler_params=_SC_PARAMS,
        scratch_types=[
            pltpu.VMEM((NCH, CH), jnp.int32),
            pltpu.VMEM((NCH, CH), jnp.int32),
            pltpu.VMEM((NCH, CH), jnp.float32),
            pltpu.VMEM((CH, HD), jnp.float32),
            pltpu.VMEM((128, HD), jnp.float32),
            pltpu.VMEM_SHARED((NPAD, HD), jnp.float32),
            pltpu.SemaphoreType.DMA,
        ],
    )
    def k(src_h, dst_h, norm_h, rh_h, out_h,
          srcb, dstb, normb, rows, zb, acc_sh, g0):
        cid = lax.axis_index("c")
        sid = lax.axis_index("s")

        pltpu.sync_copy(src_h.at[pl.ds(sid * NCH, NCH)], srcb)
        pltpu.sync_copy(dst_h.at[pl.ds(sid * NCH, NCH)], dstb)
        pltpu.sync_copy(norm_h.at[pl.ds(sid * NCH, NCH)], normb)

        _zero_rows(zb, 128)
        for kb in range(SHARD // 128):
            pltpu.sync_copy(zb, acc_sh.at[pl.ds(sid * SHARD + kb * 128, 128)])

        def offc(ch, _):
            for j in range(CH // 16):
                srcb[ch, pl.ds(j * 16, 16)] = srcb[ch, pl.ds(j * 16, 16)] + cid * N
            return 0
        lax.fori_loop(0, NCH, offc, 0)
        plsc.subcore_barrier()

        _agg_loop(rh_h, srcb, dstb, normb, rows, g0, acc_sh)
        plsc.subcore_barrier()
        pltpu.sync_copy(acc_sh.at[pl.ds(sid * SHARD, SHARD)],
                        out_h.at[pl.ds(cid * NPAD + sid * SHARD, SHARD)])

    return k(src2d, dst2d, norm2d, rh2)


def _tc_gates(saggx, saggh, x, h, dis2, w12, b12, w3a, b3):
    BR = 1000

    def body(sx, sh, xb, hb, db, w12r, b12r, w3ar, b3r, u_o, rh_o, cp_o):
        invd = db[...] * db[...]
        aggx = sx[...] + xb[...] * invd
        aggh = sh[...] + hb[...] * invd
        aggc = jnp.concatenate([aggx, aggh], axis=1)
        ru = jax.nn.sigmoid(
            jnp.dot(aggc, w12r[...], preferred_element_type=jnp.float32) + b12r[...])
        u_o[...] = ru[:, D:]
        rh_o[...] = ru[:, :D] * hb[...]
        cp_o[...] = jnp.dot(aggx, w3ar[...], preferred_element_type=jnp.float32) + b3r[...]

    bs_row = pl.BlockSpec((BR, D), lambda i: (i, 0))
    bs_col = pl.BlockSpec((BR, 1), lambda i: (i, 0))

    def full(shape):
        return pl.BlockSpec(shape, lambda i: (0, 0))

    return pl.pallas_call(
        body,
        grid=(N // BR,),
        in_specs=[bs_row, bs_row, bs_row, bs_row, bs_col,
                  full((2 * D, 2 * D)), full((1, 2 * D)), full((D, D)), full((1, D))],
        out_specs=[bs_row, bs_row, bs_row],
        out_shape=[jax.ShapeDtypeStruct((N, D), jnp.float32)] * 3,
    )(saggx, saggh, x, h, dis2, w12, b12, w3a, b3)


def _tc_final(srh, rh, dis2, cpart, u, h, w3b):
    BR = 1000

    def body(sb, rhb, db, cpb, ub, hb, w3br, o):
        invd = db[...] * db[...]
        aggrh = sb[...] + rhb[...] * invd
        c = jnp.tanh(cpb[...] + jnp.dot(aggrh, w3br[...],
                                        preferred_element_type=jnp.float32))
        o[...] = ub[...] * hb[...] + (1.0 - ub[...]) * c

    bs_row = pl.BlockSpec((BR, D), lambda i: (i, 0))
    bs_col = pl.BlockSpec((BR, 1), lambda i: (i, 0))

    return pl.pallas_call(
        body,
        grid=(N // BR,),
        in_specs=[bs_row, bs_row, bs_col, bs_row, bs_row, bs_row,
                  pl.BlockSpec((D, D), lambda i: (0, 0))],
        out_specs=bs_row,
        out_shape=jax.ShapeDtypeStruct((N, D), jnp.float32),
    )(srh, rh, dis2, cpart, u, h, w3b)


def kernel(x, edge_index, edge_weight, h, W1, b1, W2, b2, W3, b3):
    src = edge_index[0].astype(jnp.int32)
    dst = edge_index[1].astype(jnp.int32)
    w = edge_weight.astype(jnp.float32)
    pad = EPAD - src.shape[0]
    src2 = jnp.concatenate([src, jnp.zeros((pad,), jnp.int32)]).reshape(NROW2D, CH)
    dst2 = jnp.concatenate([dst, jnp.zeros((pad,), jnp.int32)]).reshape(NROW2D, CH)
    w2 = jnp.concatenate([w, jnp.zeros((pad,), jnp.float32)]).reshape(NROW2D, CH)
    feat4 = jnp.concatenate([x[:, :HD], x[:, HD:], h[:, :HD], h[:, HD:]], axis=0)

    out_a, norm2, dis = _sc_pass_a(src2, dst2, w2, feat4)
    saggx = jnp.concatenate([out_a[0:N], out_a[NPAD:NPAD + N]], axis=1)
    saggh = jnp.concatenate([out_a[2 * NPAD:2 * NPAD + N],
                             out_a[3 * NPAD:3 * NPAD + N]], axis=1)
    dis2 = dis[0:N].reshape(N, 1)

    w12 = jnp.concatenate([W1, W2], axis=1)
    b12 = jnp.concatenate([b1, b2]).reshape(1, 2 * D)
    u, rh, cpart = _tc_gates(saggx, saggh, x, h, dis2, w12, b12,
                             W3[:D], b3.reshape(1, D))

    rh2 = jnp.concatenate([rh[:, :HD], rh[:, HD:]], axis=0)
    out_b, = _sc_pass_b(src2, dst2, norm2, rh2)
    srh = jnp.concatenate([out_b[0:N], out_b[NPAD:NPAD + N]], axis=1)
    return _tc_final(srh, rh, dis2, cpart, u, h, W3[D:])
